# baseline (speedup 1.0000x reference)
import functools

import jax
import jax.numpy as jnp
from jax import lax
from jax.experimental import pallas as pl
from jax.experimental.pallas import tpu as pltpu
from jax.experimental.pallas import tpu_sc as plsc

_VOCAB = 1000000
_DIM = 64
_HIDDEN = 2000
_LABELS = 1000
_B = 4096
_L = 200
_EPS = 1e-5

_NC = 2
_NS = 16
_NW = _NC * _NS
_BPW = _B // _NW
_FIRE = 8


_HL = _L // 2


def _sc_gather_pool(content3, emb):
  mesh = plsc.VectorSubcoreMesh(
      core_axis_name="c", subcore_axis_name="s", num_cores=_NC,
      num_subcores=_NS)

  @functools.partial(
      pl.kernel,
      out_type=jax.ShapeDtypeStruct((_B, _DIM), jnp.float32),
      mesh=mesh,
      compiler_params=pltpu.CompilerParams(use_tc_tiling_on_sc=False),
      scratch_types=[
          pltpu.VMEM((_BPW, 2, _HL), jnp.int32),
          pltpu.VMEM((2, _L, _DIM), jnp.float32),
          pltpu.VMEM((_BPW, _DIM), jnp.float32),
          (pltpu.SemaphoreType.DMA, pltpu.SemaphoreType.DMA),
      ],
  )
  def body(content_hbm, emb_hbm, out_hbm, idx_v, buf_v, stage_v, sems):
    wid = lax.axis_index("s") * _NC + lax.axis_index("c")
    base = wid * _BPW
    pltpu.sync_copy(content_hbm.at[pl.ds(base, _BPW)], idx_v)

    def fire(e, slot):
      for h in range(2):
        pltpu.async_copy(
            emb_hbm.at[idx_v.at[e, h]],
            buf_v.at[slot, pl.ds(h * _HL, _HL)],
            sems[slot])

    def drain(e, slot):
      for h in range(2):
        pltpu.make_async_copy(
            emb_hbm.at[idx_v.at[e, h]],
            buf_v.at[slot, pl.ds(h * _HL, _HL)],
            sems[slot]).wait()

    def accum(e, slot):
      accs = [jnp.zeros((16,), jnp.float32) for _ in range(4)]
      for j in range(_L):
        for k in range(4):
          accs[k] = accs[k] + buf_v[slot, j, pl.ds(k * 16, 16)]
      for k in range(4):
        stage_v[e, pl.ds(k * 16, 16)] = accs[k]

    fire(0, 0)

    def step(t, carry):
      e0 = 2 * t
      fire(e0 + 1, 1)
      drain(e0, 0)
      accum(e0, 0)

      @pl.when(t < _BPW // 2 - 1)
      def _():
        fire(e0 + 2, 0)

      drain(e0 + 1, 1)
      accum(e0 + 1, 1)
      return carry

    lax.fori_loop(0, _BPW // 2, step, 0, unroll=False)

    pltpu.sync_copy(stage_v, out_hbm.at[pl.ds(base, _BPW)])

  return body(content3, emb)


_BB = 512


def _mlp_body(csum_ref, w1_ref, gamma_ref, beta_ref, w2_ref, b2_ref, out_ref,
              s_ref, m_ref):
  i = pl.program_id(0)

  @pl.when(i == 0)
  def _():
    c = csum_ref[...] * (1.0 / _L)
    m = jnp.mean(c, axis=0)
    g = lax.dot_general(c, c, (((0,), (0,)), ((), ())),
                        preferred_element_type=jnp.float32) / _B
    cov = g - m[:, None] * m[None, :]
    a = lax.dot_general(w1_ref[...], cov, (((1,), (0,)), ((), ())),
                        preferred_element_type=jnp.float32)
    var = jnp.sum(a * w1_ref[...], axis=1)
    s_ref[...] = (gamma_ref[...] * lax.rsqrt(var + _EPS)[None, :])
    m_ref[...] = m[None, :]

  blk = csum_ref[pl.ds(i * _BB, _BB), :] * (1.0 / _L) - m_ref[...]
  h = lax.dot_general(blk, w1_ref[...], (((1,), (1,)), ((), ())),
                      preferred_element_type=jnp.float32)
  r = jnp.maximum(h * s_ref[...] + beta_ref[...], 0.0)
  out_ref[...] = lax.dot_general(r, w2_ref[...], (((1,), (1,)), ((), ())),
                                 preferred_element_type=jnp.float32) + b2_ref[...]


def _tc_mlp(csum, w1, gamma, beta, w2, b2):
  grid = (_B // _BB,)
  full = lambda shape: pl.BlockSpec(shape, lambda i: (0, 0))
  return pl.pallas_call(
      _mlp_body,
      grid=grid,
      in_specs=[
          full((_B, _DIM)),
          full((_HIDDEN, _DIM)),
          full((1, _HIDDEN)),
          full((1, _HIDDEN)),
          full((_LABELS, _HIDDEN)),
          full((1, _LABELS)),
      ],
      out_specs=pl.BlockSpec((_BB, _LABELS), lambda i: (i, 0)),
      out_shape=jax.ShapeDtypeStruct((_B, _LABELS), jnp.float32),
      scratch_shapes=[
          pltpu.VMEM((1, _HIDDEN), jnp.float32),
          pltpu.VMEM((1, _DIM), jnp.float32),
      ],
  )(csum, w1, gamma.reshape(1, _HIDDEN), beta.reshape(1, _HIDDEN), w2,
    b2.reshape(1, _LABELS))


def kernel(content, emb, W1, b1, gamma, beta, W2, b2):
  del b1
  content3 = content.astype(jnp.int32).reshape(_B, 2, _HL)
  csum = _sc_gather_pool(content3, emb)
  return _tc_mlp(csum, W1, gamma, beta, W2, b2)


# speedup vs baseline: 1.4106x; 1.4106x over previous
import functools

import jax
import jax.numpy as jnp
from jax import lax
from jax.experimental import pallas as pl
from jax.experimental.pallas import tpu as pltpu
from jax.experimental.pallas import tpu_sc as plsc

_VOCAB = 1000000
_DIM = 64
_HIDDEN = 2000
_LABELS = 1000
_B = 4096
_L = 200
_EPS = 1e-5

_NC = 2
_NS = 16
_NW = _NC * _NS
_BPW = _B // _NW
_FIRE = 8


_HL = _L // 2
_PD = 128
_NBUF = 3


def _sc_gather_pool(content3, emb128):
  mesh = plsc.VectorSubcoreMesh(
      core_axis_name="c", subcore_axis_name="s", num_cores=_NC,
      num_subcores=_NS)

  @functools.partial(
      pl.kernel,
      out_type=jax.ShapeDtypeStruct((_B, _DIM), jnp.float32),
      mesh=mesh,
      scratch_types=[
          pltpu.VMEM((_BPW, 2, _HL), jnp.int32),
          pltpu.VMEM((_NBUF, _L, _PD), jnp.float32),
          pltpu.VMEM((_BPW, _DIM), jnp.float32),
          tuple(pltpu.SemaphoreType.DMA for _ in range(_NBUF)),
      ],
  )
  def body(content_hbm, emb_hbm, out_hbm, idx_v, buf_v, stage_v, sems):
    wid = lax.axis_index("s") * _NC + lax.axis_index("c")
    base = wid * _BPW
    pltpu.sync_copy(content_hbm.at[pl.ds(base, _BPW)], idx_v)

    def fire(e, slot):
      for h in range(2):
        pltpu.async_copy(
            emb_hbm.at[idx_v.at[e, h]],
            buf_v.at[slot, pl.ds(h * _HL, _HL)],
            sems[slot])

    def drain(e, slot):
      for h in range(2):
        pltpu.make_async_copy(
            emb_hbm.at[idx_v.at[e, h]],
            buf_v.at[slot, pl.ds(h * _HL, _HL)],
            sems[slot]).wait()

    def accum(e, slot):
      def jchunk(jc, accs):
        out = list(accs)
        for jj in range(8):
          j = jc * 8 + jj
          for k in range(4):
            out[k] = out[k] + buf_v[slot, j, pl.ds(k * 16, 16)]
        return tuple(out)

      accs = lax.fori_loop(
          0, _L // 8, jchunk,
          tuple(jnp.zeros((16,), jnp.float32) for _ in range(4)),
          unroll=False)
      for k in range(4):
        stage_v[e, pl.ds(k * 16, 16)] = accs[k]

    for s in range(_NBUF):
      fire(s, s)

    def step(t, carry):
      for u in range(_NBUF):
        e = _NBUF * t + u
        drain(e, u)
        accum(e, u)

        @pl.when(e + _NBUF < _BPW)
        def _():
          fire(e + _NBUF, u)

      return carry

    n_full = _BPW // _NBUF
    lax.fori_loop(0, n_full, step, 0, unroll=False)
    for e in range(n_full * _NBUF, _BPW):
      drain(e, e % _NBUF)
      accum(e, e % _NBUF)

    pltpu.sync_copy(stage_v, out_hbm.at[pl.ds(base, _BPW)])

  return body(content3, emb128)


_BB = 512


def _mlp_body(csum_ref, w1_ref, gamma_ref, beta_ref, w2_ref, b2_ref, out_ref,
              s_ref, m_ref):
  i = pl.program_id(0)

  @pl.when(i == 0)
  def _():
    c = csum_ref[...] * (1.0 / _L)
    m = jnp.mean(c, axis=0)
    g = lax.dot_general(c, c, (((0,), (0,)), ((), ())),
                        preferred_element_type=jnp.float32) / _B
    cov = g - m[:, None] * m[None, :]
    a = lax.dot_general(w1_ref[...], cov, (((1,), (0,)), ((), ())),
                        preferred_element_type=jnp.float32)
    var = jnp.sum(a * w1_ref[...], axis=1)
    s_ref[...] = (gamma_ref[...] * lax.rsqrt(var + _EPS)[None, :])
    m_ref[...] = m[None, :]

  blk = csum_ref[pl.ds(i * _BB, _BB), :] * (1.0 / _L) - m_ref[...]
  h = lax.dot_general(blk, w1_ref[...], (((1,), (1,)), ((), ())),
                      preferred_element_type=jnp.float32)
  r = jnp.maximum(h * s_ref[...] + beta_ref[...], 0.0)
  out_ref[...] = lax.dot_general(r, w2_ref[...], (((1,), (1,)), ((), ())),
                                 preferred_element_type=jnp.float32) + b2_ref[...]


def _tc_mlp(csum, w1, gamma, beta, w2, b2):
  grid = (_B // _BB,)
  full = lambda shape: pl.BlockSpec(shape, lambda i: (0, 0))
  return pl.pallas_call(
      _mlp_body,
      grid=grid,
      in_specs=[
          full((_B, _DIM)),
          full((_HIDDEN, _DIM)),
          full((1, _HIDDEN)),
          full((1, _HIDDEN)),
          full((_LABELS, _HIDDEN)),
          full((1, _LABELS)),
      ],
      out_specs=pl.BlockSpec((_BB, _LABELS), lambda i: (i, 0)),
      out_shape=jax.ShapeDtypeStruct((_B, _LABELS), jnp.float32),
      scratch_shapes=[
          pltpu.VMEM((1, _HIDDEN), jnp.float32),
          pltpu.VMEM((1, _DIM), jnp.float32),
      ],
  )(csum, w1, gamma.reshape(1, _HIDDEN), beta.reshape(1, _HIDDEN), w2,
    b2.reshape(1, _LABELS))


def kernel(content, emb, W1, b1, gamma, beta, W2, b2):
  del b1
  content3 = content.astype(jnp.int32).reshape(_B, 2, _HL)
  emb128 = jnp.pad(emb, ((0, 0), (0, _PD - _DIM)))
  csum = _sc_gather_pool(content3, emb128)
  return _tc_mlp(csum, W1, gamma, beta, W2, b2)


# speedup vs baseline: 1.5337x; 1.0873x over previous
import functools

import jax
import jax.numpy as jnp
from jax import lax
from jax.experimental import pallas as pl
from jax.experimental.pallas import tpu as pltpu
from jax.experimental.pallas import tpu_sc as plsc

_VOCAB = 1000000
_DIM = 64
_HIDDEN = 2000
_LABELS = 1000
_B = 4096
_L = 200
_EPS = 1e-5

_NC = 2
_NS = 16
_NW = _NC * _NS
_BPW = _B // _NW
_FIRE = 8


_HL = _L // 2
_PD = 128
_NBUF = 3


def _sc_gather_pool(content3, emb128):
  mesh = plsc.VectorSubcoreMesh(
      core_axis_name="c", subcore_axis_name="s", num_cores=_NC,
      num_subcores=_NS)

  @functools.partial(
      pl.kernel,
      out_type=jax.ShapeDtypeStruct((_B, _DIM), jnp.float32),
      mesh=mesh,
      scratch_types=[
          pltpu.VMEM((_BPW, 2, _HL), jnp.int32),
          pltpu.VMEM((_NBUF, _L, _PD), jnp.float32),
          pltpu.VMEM((_BPW, _DIM), jnp.float32),
          tuple(pltpu.SemaphoreType.DMA for _ in range(_NBUF)),
      ],
  )
  def body(content_hbm, emb_hbm, out_hbm, idx_v, buf_v, stage_v, sems):
    wid = lax.axis_index("s") * _NC + lax.axis_index("c")
    base = wid * _BPW
    pltpu.sync_copy(content_hbm.at[pl.ds(base, _BPW)], idx_v)

    def fire(e, slot):
      for h in range(2):
        pltpu.async_copy(
            emb_hbm.at[idx_v.at[e, h]],
            buf_v.at[slot, pl.ds(h * _HL, _HL)],
            sems[slot])

    def drain(e, slot):
      for h in range(2):
        pltpu.make_async_copy(
            emb_hbm.at[idx_v.at[e, h]],
            buf_v.at[slot, pl.ds(h * _HL, _HL)],
            sems[slot]).wait()

    def accum(e, slot):
      def jchunk(jc, accs):
        out = list(accs)
        for jj in range(8):
          j = jc * 8 + jj
          for k in range(4):
            out[k] = out[k] + buf_v[slot, j, pl.ds(k * 16, 16)]
        return tuple(out)

      accs = lax.fori_loop(
          0, _L // 8, jchunk,
          tuple(jnp.zeros((16,), jnp.float32) for _ in range(4)),
          unroll=False)
      for k in range(4):
        stage_v[e, pl.ds(k * 16, 16)] = accs[k]

    for s in range(_NBUF):
      fire(s, s)

    def step(t, carry):
      for u in range(_NBUF):
        e = _NBUF * t + u
        drain(e, u)
        accum(e, u)

        @pl.when(e + _NBUF < _BPW)
        def _():
          fire(e + _NBUF, u)

      return carry

    n_full = _BPW // _NBUF
    lax.fori_loop(0, n_full, step, 0, unroll=False)
    for e in range(n_full * _NBUF, _BPW):
      drain(e, e % _NBUF)
      accum(e, e % _NBUF)

    pltpu.sync_copy(stage_v, out_hbm.at[pl.ds(base, _BPW)])

  return body(content3, emb128)


_VB = 2048


def _transpose_body(in_ref, out_ref):
  xt = in_ref[...].T
  out_ref[...] = jnp.concatenate([xt, jnp.zeros_like(xt)], axis=1)


def _tc_table_transpose(emb_t):
  return pl.pallas_call(
      _transpose_body,
      grid=(pl.cdiv(_VOCAB, _VB),),
      in_specs=[pl.BlockSpec((_DIM, _VB), lambda i: (0, i))],
      out_specs=pl.BlockSpec((_VB, _PD), lambda i: (i, 0)),
      out_shape=jax.ShapeDtypeStruct((_VOCAB, _PD), jnp.float32),
  )(emb_t)


_BB = 512


def _mlp_body(csum_ref, w1_ref, gamma_ref, beta_ref, w2_ref, b2_ref, out_ref,
              s_ref, m_ref):
  i = pl.program_id(0)

  @pl.when(i == 0)
  def _():
    c = csum_ref[...] * (1.0 / _L)
    m = jnp.mean(c, axis=0)
    g = lax.dot_general(c, c, (((0,), (0,)), ((), ())),
                        preferred_element_type=jnp.float32) / _B
    cov = g - m[:, None] * m[None, :]
    a = lax.dot_general(w1_ref[...], cov, (((1,), (0,)), ((), ())),
                        preferred_element_type=jnp.float32)
    var = jnp.sum(a * w1_ref[...], axis=1)
    s_ref[...] = (gamma_ref[...] * lax.rsqrt(var + _EPS)[None, :])
    m_ref[...] = m[None, :]

  blk = csum_ref[pl.ds(i * _BB, _BB), :] * (1.0 / _L) - m_ref[...]
  h = lax.dot_general(blk, w1_ref[...], (((1,), (1,)), ((), ())),
                      preferred_element_type=jnp.float32)
  r = jnp.maximum(h * s_ref[...] + beta_ref[...], 0.0)
  out_ref[...] = lax.dot_general(r, w2_ref[...], (((1,), (1,)), ((), ())),
                                 preferred_element_type=jnp.float32) + b2_ref[...]


def _tc_mlp(csum, w1, gamma, beta, w2, b2):
  grid = (_B // _BB,)
  full = lambda shape: pl.BlockSpec(shape, lambda i: (0, 0))
  return pl.pallas_call(
      _mlp_body,
      grid=grid,
      in_specs=[
          full((_B, _DIM)),
          full((_HIDDEN, _DIM)),
          full((1, _HIDDEN)),
          full((1, _HIDDEN)),
          full((_LABELS, _HIDDEN)),
          full((1, _LABELS)),
      ],
      out_specs=pl.BlockSpec((_BB, _LABELS), lambda i: (i, 0)),
      out_shape=jax.ShapeDtypeStruct((_B, _LABELS), jnp.float32),
      scratch_shapes=[
          pltpu.VMEM((1, _HIDDEN), jnp.float32),
          pltpu.VMEM((1, _DIM), jnp.float32),
      ],
  )(csum, w1, gamma.reshape(1, _HIDDEN), beta.reshape(1, _HIDDEN), w2,
    b2.reshape(1, _LABELS))


def kernel(content, emb, W1, b1, gamma, beta, W2, b2):
  del b1
  content3 = content.astype(jnp.int32).reshape(_B, 2, _HL)
  emb128 = _tc_table_transpose(emb.T)
  csum = _sc_gather_pool(content3, emb128)
  return _tc_mlp(csum, W1, gamma, beta, W2, b2)
